# baseline (speedup 1.0000x reference)
import functools

import jax
import jax.numpy as jnp
from jax import lax
from jax.experimental import pallas as pl
from jax.experimental.pallas import tpu as pltpu
from jax.experimental.pallas import tpu_sc as plsc

N = 10000
D = 128
HD = 64
E = 320000
TB = 4096
NW = 32
CK = 128
CHD = 80
CHE = 160
G = 2
NCH = CHE // G
NPAD = 10016
RPT = NPAD // 16

_mesh = plsc.VectorSubcoreMesh(core_axis_name="c", subcore_axis_name="s")
_sc_params = pltpu.CompilerParams(use_tc_tiling_on_sc=False)


@functools.partial(
    pl.kernel, mesh=_mesh, compiler_params=_sc_params,
    out_type=jax.ShapeDtypeStruct((NW, RPT, 16), jnp.float32),
    scratch_types=[
        pltpu.VMEM((CHD, CK), jnp.int32),
        pltpu.VMEM((CK, 16), jnp.float32),
        pltpu.VMEM_SHARED((NPAD, 16), jnp.float32),
    ],
)
def _deg_kernel(dst_hbm, ones_hbm, zero_hbm, out_hbm, dstv, onesv, acc):
    c = lax.axis_index("c")
    s = lax.axis_index("s")
    w = c * 16 + s
    pltpu.sync_copy(dst_hbm.at[w], dstv)
    pltpu.sync_copy(ones_hbm, onesv)
    pltpu.sync_copy(zero_hbm, acc.at[pl.ds(s * RPT, RPT)])
    plsc.subcore_barrier()

    def body(i, carry):
        pltpu.sync_copy(onesv, acc.at[dstv.at[i]], add=True)
        return carry

    lax.fori_loop(0, CHD, body, 0)
    plsc.subcore_barrier()
    pltpu.sync_copy(acc.at[pl.ds(s * RPT, RPT)], out_hbm.at[w])


@functools.partial(
    pl.kernel, mesh=_mesh, compiler_params=_sc_params,
    out_type=jax.ShapeDtypeStruct((NW, RPT, HD), jnp.float32),
    scratch_types=[
        pltpu.VMEM((NCH, G * CK), jnp.int32),
        pltpu.VMEM((NCH, G * CK), jnp.int32),
        pltpu.VMEM((2, G * CK, HD), jnp.float32),
        pltpu.VMEM_SHARED((NPAD, HD), jnp.float32),
        pltpu.SemaphoreType.DMA,
        pltpu.SemaphoreType.DMA,
    ],
)
def _edge_kernel(hp_hbm, src_hbm, dst_hbm, zero_hbm, out_hbm,
                 srcv, dstv, rows, acc, sem0, sem1):
    c = lax.axis_index("c")
    s = lax.axis_index("s")
    w = c * 16 + s
    pltpu.sync_copy(src_hbm.at[w], srcv)
    pltpu.sync_copy(dst_hbm.at[w], dstv)
    pltpu.sync_copy(zero_hbm, acc.at[pl.ds(s * RPT, RPT)])
    plsc.subcore_barrier()

    def g(ci, buf, sem):
        return pltpu.make_async_copy(hp_hbm.at[srcv.at[ci]], rows.at[buf], sem)

    g(0, 0, sem0).start()
    g(1, 1, sem1).start()

    def body(k, carry):
        i = k * 2
        g(i, 0, sem0).wait()
        pltpu.sync_copy(rows.at[0], acc.at[dstv.at[i]], add=True)

        @pl.when(i + 2 < NCH)
        def _s0():
            g(i + 2, 0, sem0).start()

        g(i + 1, 1, sem1).wait()
        pltpu.sync_copy(rows.at[1], acc.at[dstv.at[i + 1]], add=True)

        @pl.when(i + 3 < NCH)
        def _s1():
            g(i + 3, 1, sem1).start()

        return carry

    lax.fori_loop(0, NCH // 2, body, 0)
    plsc.subcore_barrier()
    pltpu.sync_copy(acc.at[pl.ds(s * RPT, RPT)], out_hbm.at[w])


@functools.partial(
    pl.kernel, mesh=_mesh, compiler_params=_sc_params,
    out_type=jax.ShapeDtypeStruct((TB, D), jnp.float32),
    scratch_types=[
        pltpu.VMEM((TB // NW,), jnp.int32),
        pltpu.VMEM((TB // NW, D), jnp.float32),
        pltpu.SemaphoreType.DMA,
    ],
)
def _target_gather(y_hbm, ti_hbm, out_hbm, idxv, rowsv, sem):
    c = lax.axis_index("c")
    s = lax.axis_index("s")
    w = c * 16 + s
    bw = TB // NW
    pltpu.sync_copy(ti_hbm.at[pl.ds(w * bw, bw)], idxv)
    cp = pltpu.make_async_copy(y_hbm.at[idxv], rowsv, sem)
    cp.start()
    cp.wait()
    pltpu.sync_copy(rowsv, out_hbm.at[pl.ds(w * bw, bw)])


def _split(h, tab_ref):
    tab_ref[:NPAD] = h[:, :HD]
    tab_ref[NPAD:] = h[:, HD:]


def _join(tab_ref):
    return jnp.concatenate([tab_ref[:NPAD], tab_ref[NPAD:]], axis=1)


def _prep0_body(degp, xp, w0, htab, dinv):
    d = jnp.sum(degp[0] + degp[1], axis=1)
    r = lax.broadcasted_iota(jnp.int32, (NPAD, 1), 0)
    di = lax.rsqrt(d + 1.0)[:, None]
    di = jnp.where(r < N, di, 0.0)
    di2 = jnp.broadcast_to(di, (NPAD, D))
    _split((xp[...] @ w0[...]) * di2, htab)
    dinv[...] = di2


def _mid_body(p, htab0, dinv, b0, g0, be0, w1, htab1):
    adj = jnp.concatenate([p[0], p[1]], axis=1)
    s0 = (adj + _join(htab0)) * dinv[...] + b0[...]
    sr = s0[:N]
    mean = jnp.mean(sr, axis=0, keepdims=True)
    var = jnp.mean((sr - mean) ** 2, axis=0, keepdims=True)
    h = (s0 - mean) * lax.rsqrt(var + 1e-5) * g0[...] + be0[...]
    h = jnp.maximum(h, 0.0)
    _split((h @ w1[...]) * dinv[...], htab1)


def _fin_body(p, htab1, dinv, b1, wf1, bf1, wf2, bf2, y):
    adj = jnp.concatenate([p[0], p[1]], axis=1)
    out1 = (adj + _join(htab1)) * dinv[...] + b1[...]
    f = jnp.maximum(out1 @ wf1[...] + bf1[...], 0.0)
    y[...] = f @ wf2[...] + bf2[...]


def kernel(x, edge_index, target_index, W0, b0, gamma0, beta0,
           W1, b1, Wf1, bf1, Wf2, bf2):
    f32 = jnp.float32
    src = edge_index[0].astype(jnp.int32)
    dst = edge_index[1].astype(jnp.int32)

    padd = jnp.full((NW * CHD * CK - E,), N, jnp.int32)
    dst_wd = jnp.concatenate([dst, padd]).reshape(NW, CHD, CK)

    pade = jnp.full((16 * CHE * CK - E,), N, jnp.int32)
    src_h = jnp.concatenate([src, pade]).reshape(16, NCH, G * CK)
    src_we = jnp.concatenate([src_h, src_h + NPAD], axis=0)
    dst_h = jnp.concatenate([dst, pade]).reshape(16, NCH, G * CK)
    dst_we = jnp.concatenate([dst_h, dst_h], axis=0)

    ti = target_index.astype(jnp.int32)
    xpad = jnp.concatenate([x.astype(f32), jnp.zeros((NPAD - N, D), f32)], 0)
    ones16 = jnp.zeros((CK, 16), f32).at[:, 0].set(1.0)
    zdeg = jnp.zeros((RPT, 16), f32)
    zrow = jnp.zeros((RPT, HD), f32)

    degp = _deg_kernel(dst_wd, ones16, zdeg).reshape(2, NPAD, 16)

    htab0, dinv = pl.pallas_call(
        _prep0_body,
        out_shape=[jax.ShapeDtypeStruct((2 * NPAD, HD), f32),
                   jax.ShapeDtypeStruct((NPAD, D), f32)],
    )(degp, xpad, W0)

    p0 = _edge_kernel(htab0, src_we, dst_we, zrow).reshape(2, NPAD, HD)

    htab1 = pl.pallas_call(
        _mid_body,
        out_shape=jax.ShapeDtypeStruct((2 * NPAD, HD), f32),
    )(p0, htab0, dinv, b0.reshape(1, D), gamma0.reshape(1, D),
      beta0.reshape(1, D), W1)

    p1 = _edge_kernel(htab1, src_we, dst_we, zrow).reshape(2, NPAD, HD)

    y = pl.pallas_call(
        _fin_body,
        out_shape=jax.ShapeDtypeStruct((NPAD, D), f32),
    )(p1, htab1, dinv, b1.reshape(1, D), Wf1, bf1.reshape(1, D),
      Wf2, bf2.reshape(1, D))

    return _target_gather(y, ti)


# speedup vs baseline: 1.0233x; 1.0233x over previous
import functools

import jax
import jax.numpy as jnp
from jax import lax
from jax.experimental import pallas as pl
from jax.experimental.pallas import tpu as pltpu
from jax.experimental.pallas import tpu_sc as plsc

N = 10000
D = 128
HD = 64
E = 320000
TB = 4096
NW = 32
CK = 128
CHD = 80
CHE = 160
NPAD = 10016
RPT = NPAD // 16

_mesh = plsc.VectorSubcoreMesh(core_axis_name="c", subcore_axis_name="s")
_sc_params = pltpu.CompilerParams(use_tc_tiling_on_sc=False)


@functools.partial(
    pl.kernel, mesh=_mesh, compiler_params=_sc_params,
    out_type=jax.ShapeDtypeStruct((NW, RPT, 16), jnp.float32),
    scratch_types=[
        pltpu.VMEM((CHD, CK), jnp.int32),
        pltpu.VMEM((CK, 16), jnp.float32),
        pltpu.VMEM_SHARED((NPAD, 16), jnp.float32),
    ],
)
def _deg_kernel(dst_hbm, ones_hbm, zero_hbm, out_hbm, dstv, onesv, acc):
    c = lax.axis_index("c")
    s = lax.axis_index("s")
    w = c * 16 + s
    pltpu.sync_copy(dst_hbm.at[w], dstv)
    pltpu.sync_copy(ones_hbm, onesv)
    pltpu.sync_copy(zero_hbm, acc.at[pl.ds(s * RPT, RPT)])
    plsc.subcore_barrier()

    def body(i, carry):
        pltpu.sync_copy(onesv, acc.at[dstv.at[i]], add=True)
        return carry

    lax.fori_loop(0, CHD, body, 0)
    plsc.subcore_barrier()
    pltpu.sync_copy(acc.at[pl.ds(s * RPT, RPT)], out_hbm.at[w])


@functools.partial(
    pl.kernel, mesh=_mesh, compiler_params=_sc_params,
    out_type=jax.ShapeDtypeStruct((NW, RPT, HD), jnp.float32),
    scratch_types=[
        pltpu.VMEM((CHE, CK), jnp.int32),
        pltpu.VMEM((CHE, CK), jnp.int32),
        pltpu.VMEM((2, CK, HD), jnp.float32),
        pltpu.VMEM_SHARED((NPAD, HD), jnp.float32),
        pltpu.SemaphoreType.DMA,
        pltpu.SemaphoreType.DMA,
    ],
)
def _edge_kernel(hp_hbm, src_hbm, dst_hbm, zero_hbm, out_hbm,
                 srcv, dstv, rows, acc, sem0, sem1):
    c = lax.axis_index("c")
    s = lax.axis_index("s")
    w = c * 16 + s
    pltpu.sync_copy(src_hbm.at[w], srcv)
    pltpu.sync_copy(dst_hbm.at[w], dstv)
    pltpu.sync_copy(zero_hbm, acc.at[pl.ds(s * RPT, RPT)])
    plsc.subcore_barrier()

    def g(ci, buf, sem):
        return pltpu.make_async_copy(hp_hbm.at[srcv.at[ci]], rows.at[buf], sem)

    g(0, 0, sem0).start()
    g(1, 1, sem1).start()

    def body(k, carry):
        i = k * 2
        g(i, 0, sem0).wait()
        pltpu.sync_copy(rows.at[0], acc.at[dstv.at[i]], add=True)

        @pl.when(i + 2 < CHE)
        def _s0():
            g(i + 2, 0, sem0).start()

        g(i + 1, 1, sem1).wait()
        pltpu.sync_copy(rows.at[1], acc.at[dstv.at[i + 1]], add=True)

        @pl.when(i + 3 < CHE)
        def _s1():
            g(i + 3, 1, sem1).start()

        return carry

    lax.fori_loop(0, CHE // 2, body, 0)
    plsc.subcore_barrier()
    pltpu.sync_copy(acc.at[pl.ds(s * RPT, RPT)], out_hbm.at[w])


@functools.partial(
    pl.kernel, mesh=_mesh, compiler_params=_sc_params,
    out_type=jax.ShapeDtypeStruct((TB, D), jnp.float32),
    scratch_types=[
        pltpu.VMEM((TB // NW,), jnp.int32),
        pltpu.VMEM((TB // NW, D), jnp.float32),
        pltpu.SemaphoreType.DMA,
    ],
)
def _target_gather(y_hbm, ti_hbm, out_hbm, idxv, rowsv, sem):
    c = lax.axis_index("c")
    s = lax.axis_index("s")
    w = c * 16 + s
    bw = TB // NW
    pltpu.sync_copy(ti_hbm.at[pl.ds(w * bw, bw)], idxv)
    cp = pltpu.make_async_copy(y_hbm.at[idxv], rowsv, sem)
    cp.start()
    cp.wait()
    pltpu.sync_copy(rowsv, out_hbm.at[pl.ds(w * bw, bw)])


def _split(h, tab_ref):
    tab_ref[:NPAD] = h[:, :HD]
    tab_ref[NPAD:] = h[:, HD:]


def _join(tab_ref):
    return jnp.concatenate([tab_ref[:NPAD], tab_ref[NPAD:]], axis=1)


def _prep0_body(degp, xp, w0, htab, dinv):
    d = jnp.sum(degp[0] + degp[1], axis=1)
    r = lax.broadcasted_iota(jnp.int32, (NPAD, 1), 0)
    di = lax.rsqrt(d + 1.0)[:, None]
    di = jnp.where(r < N, di, 0.0)
    di2 = jnp.broadcast_to(di, (NPAD, D))
    _split((xp[...] @ w0[...]) * di2, htab)
    dinv[...] = di2


def _mid_body(p, htab0, dinv, b0, g0, be0, w1, htab1):
    adj = jnp.concatenate([p[0], p[1]], axis=1)
    s0 = (adj + _join(htab0)) * dinv[...] + b0[...]
    sr = s0[:N]
    mean = jnp.mean(sr, axis=0, keepdims=True)
    var = jnp.mean((sr - mean) ** 2, axis=0, keepdims=True)
    h = (s0 - mean) * lax.rsqrt(var + 1e-5) * g0[...] + be0[...]
    h = jnp.maximum(h, 0.0)
    _split((h @ w1[...]) * dinv[...], htab1)


def _fin_body(p, htab1, dinv, b1, wf1, bf1, wf2, bf2, y):
    adj = jnp.concatenate([p[0], p[1]], axis=1)
    out1 = (adj + _join(htab1)) * dinv[...] + b1[...]
    f = jnp.maximum(out1 @ wf1[...] + bf1[...], 0.0)
    y[...] = f @ wf2[...] + bf2[...]


def kernel(x, edge_index, target_index, W0, b0, gamma0, beta0,
           W1, b1, Wf1, bf1, Wf2, bf2):
    f32 = jnp.float32
    src = edge_index[0].astype(jnp.int32)
    dst = edge_index[1].astype(jnp.int32)

    padd = jnp.full((NW * CHD * CK - E,), N, jnp.int32)
    dst_wd = jnp.concatenate([dst, padd]).reshape(NW, CHD, CK)

    pade = jnp.full((16 * CHE * CK - E,), N, jnp.int32)
    src_h = jnp.concatenate([src, pade]).reshape(16, CHE, CK)
    src_we = jnp.concatenate([src_h, src_h + NPAD], axis=0)
    dst_h = jnp.concatenate([dst, pade]).reshape(16, CHE, CK)
    dst_we = jnp.concatenate([dst_h, dst_h], axis=0)

    ti = target_index.astype(jnp.int32)
    xpad = jnp.concatenate([x.astype(f32), jnp.zeros((NPAD - N, D), f32)], 0)
    ones16 = jnp.zeros((CK, 16), f32).at[:, 0].set(1.0)
    zdeg = jnp.zeros((RPT, 16), f32)
    zrow = jnp.zeros((RPT, HD), f32)

    degp = _deg_kernel(dst_wd, ones16, zdeg).reshape(2, NPAD, 16)

    htab0, dinv = pl.pallas_call(
        _prep0_body,
        out_shape=[jax.ShapeDtypeStruct((2 * NPAD, HD), f32),
                   jax.ShapeDtypeStruct((NPAD, D), f32)],
    )(degp, xpad, W0)

    p0 = _edge_kernel(htab0, src_we, dst_we, zrow).reshape(2, NPAD, HD)

    htab1 = pl.pallas_call(
        _mid_body,
        out_shape=jax.ShapeDtypeStruct((2 * NPAD, HD), f32),
    )(p0, htab0, dinv, b0.reshape(1, D), gamma0.reshape(1, D),
      beta0.reshape(1, D), W1)

    p1 = _edge_kernel(htab1, src_we, dst_we, zrow).reshape(2, NPAD, HD)

    y = pl.pallas_call(
        _fin_body,
        out_shape=jax.ShapeDtypeStruct((NPAD, D), f32),
    )(p1, htab1, dinv, b1.reshape(1, D), Wf1, bf1.reshape(1, D),
      Wf2, bf2.reshape(1, D))

    return _target_gather(y, ti)


# speedup vs baseline: 1.0243x; 1.0010x over previous
import functools

import jax
import jax.numpy as jnp
from jax import lax
from jax.experimental import pallas as pl
from jax.experimental.pallas import tpu as pltpu
from jax.experimental.pallas import tpu_sc as plsc

N = 10000
D = 128
HD = 64
E = 320000
TB = 4096
NW = 32
CK = 128
CHD = 80
CHE = 160
NPAD = 10016
RPT = NPAD // 16

_mesh = plsc.VectorSubcoreMesh(core_axis_name="c", subcore_axis_name="s")
_sc_params = pltpu.CompilerParams(use_tc_tiling_on_sc=False)


@functools.partial(
    pl.kernel, mesh=_mesh, compiler_params=_sc_params,
    out_type=jax.ShapeDtypeStruct((NW, RPT, 16), jnp.float32),
    scratch_types=[
        pltpu.VMEM((CHD, CK), jnp.int32),
        pltpu.VMEM((CK, 16), jnp.float32),
        pltpu.VMEM_SHARED((NPAD, 16), jnp.float32),
    ],
)
def _deg_kernel(dst_hbm, ones_hbm, zero_hbm, out_hbm, dstv, onesv, acc):
    c = lax.axis_index("c")
    s = lax.axis_index("s")
    w = c * 16 + s
    pltpu.sync_copy(dst_hbm.at[w], dstv)
    pltpu.sync_copy(ones_hbm, onesv)
    pltpu.sync_copy(zero_hbm, acc.at[pl.ds(s * RPT, RPT)])
    plsc.subcore_barrier()

    def body(i, carry):
        pltpu.sync_copy(onesv, acc.at[dstv.at[i]], add=True)
        return carry

    lax.fori_loop(0, CHD, body, 0)
    plsc.subcore_barrier()
    pltpu.sync_copy(acc.at[pl.ds(s * RPT, RPT)], out_hbm.at[w])


@functools.partial(
    pl.kernel, mesh=_mesh, compiler_params=_sc_params,
    out_type=jax.ShapeDtypeStruct((NW, RPT, HD), jnp.float32),
    scratch_types=[
        pltpu.VMEM((CHE, CK), jnp.int32),
        pltpu.VMEM((CHE, CK), jnp.int32),
        pltpu.VMEM((2, CK, HD), jnp.float32),
        pltpu.VMEM_SHARED((NPAD, HD), jnp.float32),
        pltpu.SemaphoreType.DMA,
        pltpu.SemaphoreType.DMA,
    ],
)
def _edge_kernel(hp_hbm, src_hbm, dst_hbm, zero_hbm, out_hbm,
                 srcv, dstv, rows, acc, sem0, sem1):
    c = lax.axis_index("c")
    s = lax.axis_index("s")
    w = c * 16 + s
    pltpu.sync_copy(src_hbm.at[w], srcv)
    pltpu.sync_copy(dst_hbm.at[w], dstv)
    pltpu.sync_copy(zero_hbm, acc.at[pl.ds(s * RPT, RPT)])
    plsc.subcore_barrier()

    def g(ci, buf, sem):
        return pltpu.make_async_copy(hp_hbm.at[srcv.at[ci]], rows.at[buf], sem)

    g(0, 0, sem0).start()
    g(1, 1, sem1).start()

    def body(k, carry):
        i = k * 2
        g(i, 0, sem0).wait()
        pltpu.sync_copy(rows.at[0], acc.at[dstv.at[i]], add=True)

        @pl.when(i + 2 < CHE)
        def _s0():
            g(i + 2, 0, sem0).start()

        g(i + 1, 1, sem1).wait()
        pltpu.sync_copy(rows.at[1], acc.at[dstv.at[i + 1]], add=True)

        @pl.when(i + 3 < CHE)
        def _s1():
            g(i + 3, 1, sem1).start()

        return carry

    lax.fori_loop(0, CHE // 2, body, 0)
    plsc.subcore_barrier()
    pltpu.sync_copy(acc.at[pl.ds(s * RPT, RPT)], out_hbm.at[w])


@functools.partial(
    pl.kernel, mesh=_mesh, compiler_params=_sc_params,
    out_type=jax.ShapeDtypeStruct((TB, D), jnp.float32),
    scratch_types=[
        pltpu.VMEM((TB // NW,), jnp.int32),
        pltpu.VMEM((TB // NW, D), jnp.float32),
        pltpu.SemaphoreType.DMA,
    ],
)
def _target_gather(y_hbm, ti_hbm, out_hbm, idxv, rowsv, sem):
    c = lax.axis_index("c")
    s = lax.axis_index("s")
    w = c * 16 + s
    bw = TB // NW
    pltpu.sync_copy(ti_hbm.at[pl.ds(w * bw, bw)], idxv)
    cp = pltpu.make_async_copy(y_hbm.at[idxv], rowsv, sem)
    cp.start()
    cp.wait()
    pltpu.sync_copy(rowsv, out_hbm.at[pl.ds(w * bw, bw)])


def _split(h, tab_ref):
    tab_ref[:NPAD] = h[:, :HD]
    tab_ref[NPAD:] = h[:, HD:]


def _join(tab_ref):
    return jnp.concatenate([tab_ref[:NPAD], tab_ref[NPAD:]], axis=1)


def _prep0_body(degp, xp, w0, htab, dinv):
    d = jnp.sum(degp[0] + degp[1], axis=1)
    r = lax.broadcasted_iota(jnp.int32, (NPAD, 1), 0)
    di = lax.rsqrt(d + 1.0)[:, None]
    di = jnp.where(r < N, di, 0.0)
    di2 = jnp.broadcast_to(di, (NPAD, D))
    _split((xp[...] @ w0[...]) * di2, htab)
    dinv[...] = di2


def _mid_body(p, htab0, dinv, b0, g0, be0, w1, htab1):
    adj = jnp.concatenate([p[0], p[1]], axis=1)
    s0 = (adj + _join(htab0)) * dinv[...] + b0[...]
    sr = s0[:N]
    mean = jnp.mean(sr, axis=0, keepdims=True)
    var = jnp.mean((sr - mean) ** 2, axis=0, keepdims=True)
    h = (s0 - mean) * lax.rsqrt(var + 1e-5) * g0[...] + be0[...]
    h = jnp.maximum(h, 0.0)
    _split((h @ w1[...]) * dinv[...], htab1)


def _fin_body(p, htab1, dinv, b1, wf1, bf1, wf2, bf2, y):
    adj = jnp.concatenate([p[0], p[1]], axis=1)
    out1 = (adj + _join(htab1)) * dinv[...] + b1[...]
    f = jnp.maximum(out1 @ wf1[...] + bf1[...], 0.0)
    y[...] = f @ wf2[...] + bf2[...]


def kernel(x, edge_index, target_index, W0, b0, gamma0, beta0,
           W1, b1, Wf1, bf1, Wf2, bf2):
    f32 = jnp.float32
    src = edge_index[0].astype(jnp.int32)
    dst = edge_index[1].astype(jnp.int32)

    npadd = NW * CHD * CK - E
    padd = N + (jnp.arange(npadd, dtype=jnp.int32) % (NPAD - N))
    dst_wd = jnp.concatenate([dst, padd]).reshape(NW, CHD, CK)

    npade = 16 * CHE * CK - E
    pads = jnp.full((npade,), N, jnp.int32)
    pade = jnp.arange(npade, dtype=jnp.int32) % NPAD
    src_h = jnp.concatenate([src, pads]).reshape(16, CHE, CK)
    src_we = jnp.concatenate([src_h, src_h + NPAD], axis=0)
    dst_h = jnp.concatenate([dst, pade]).reshape(16, CHE, CK)
    dst_we = jnp.concatenate([dst_h, dst_h], axis=0)

    ti = target_index.astype(jnp.int32)
    xpad = jnp.concatenate([x.astype(f32), jnp.zeros((NPAD - N, D), f32)], 0)
    ones16 = jnp.zeros((CK, 16), f32).at[:, 0].set(1.0)
    zdeg = jnp.zeros((RPT, 16), f32)
    zrow = jnp.zeros((RPT, HD), f32)

    degp = _deg_kernel(dst_wd, ones16, zdeg).reshape(2, NPAD, 16)

    htab0, dinv = pl.pallas_call(
        _prep0_body,
        out_shape=[jax.ShapeDtypeStruct((2 * NPAD, HD), f32),
                   jax.ShapeDtypeStruct((NPAD, D), f32)],
    )(degp, xpad, W0)

    p0 = _edge_kernel(htab0, src_we, dst_we, zrow).reshape(2, NPAD, HD)

    htab1 = pl.pallas_call(
        _mid_body,
        out_shape=jax.ShapeDtypeStruct((2 * NPAD, HD), f32),
    )(p0, htab0, dinv, b0.reshape(1, D), gamma0.reshape(1, D),
      beta0.reshape(1, D), W1)

    p1 = _edge_kernel(htab1, src_we, dst_we, zrow).reshape(2, NPAD, HD)

    y = pl.pallas_call(
        _fin_body,
        out_shape=jax.ShapeDtypeStruct((NPAD, D), f32),
    )(p1, htab1, dinv, b1.reshape(1, D), Wf1, bf1.reshape(1, D),
      Wf2, bf2.reshape(1, D))

    return _target_gather(y, ti)


# speedup vs baseline: 1.3432x; 1.3113x over previous
import functools

import jax
import jax.numpy as jnp
from jax import lax
from jax.experimental import pallas as pl
from jax.experimental.pallas import tpu as pltpu
from jax.experimental.pallas import tpu_sc as plsc

N = 10000
D = 128
HD = 64
E = 320000
TB = 4096
NW = 32
CK = 128
CHD = 80
CHE = 158
NPAD = 10016
RPT = NPAD // 16

_mesh = plsc.VectorSubcoreMesh(core_axis_name="c", subcore_axis_name="s")
_sc_params = pltpu.CompilerParams(use_tc_tiling_on_sc=False)


@functools.partial(
    pl.kernel, mesh=_mesh, compiler_params=_sc_params,
    out_type=jax.ShapeDtypeStruct((NW, RPT, 16), jnp.float32),
    scratch_types=[
        pltpu.VMEM((CHD, CK), jnp.int32),
        pltpu.VMEM((CK, 16), jnp.float32),
        pltpu.VMEM_SHARED((NPAD, 16), jnp.float32),
    ],
)
def _deg_kernel(dst_hbm, ones_hbm, zero_hbm, out_hbm, dstv, onesv, acc):
    c = lax.axis_index("c")
    s = lax.axis_index("s")
    w = c * 16 + s
    pltpu.sync_copy(dst_hbm.at[w], dstv)
    pltpu.sync_copy(ones_hbm, onesv)
    pltpu.sync_copy(zero_hbm, acc.at[pl.ds(s * RPT, RPT)])
    plsc.subcore_barrier()

    def body(i, carry):
        pltpu.sync_copy(onesv, acc.at[dstv.at[i]], add=True)
        return carry

    lax.fori_loop(0, CHD, body, 0)
    plsc.subcore_barrier()
    pltpu.sync_copy(acc.at[pl.ds(s * RPT, RPT)], out_hbm.at[w])


@functools.partial(
    pl.kernel, mesh=_mesh, compiler_params=_sc_params,
    out_type=jax.ShapeDtypeStruct((NW, RPT, HD), jnp.float32),
    scratch_types=[
        pltpu.VMEM((CHE, CK), jnp.int32),
        pltpu.VMEM((CHE, CK), jnp.int32),
        pltpu.VMEM((2, CK, HD), jnp.float32),
        pltpu.VMEM_SHARED((NPAD, HD), jnp.float32),
        pltpu.SemaphoreType.DMA,
        pltpu.SemaphoreType.DMA,
    ],
)
def _edge_kernel(hp_hbm, src_hbm, dst_hbm, zero_hbm, out_hbm,
                 srcv, dstv, rows, acc, sem0, sem1):
    c = lax.axis_index("c")
    s = lax.axis_index("s")
    w = c * 16 + s
    pltpu.sync_copy(src_hbm.at[w], srcv)
    pltpu.sync_copy(dst_hbm.at[w], dstv)
    pltpu.sync_copy(zero_hbm, acc.at[pl.ds(s * RPT, RPT)])
    plsc.subcore_barrier()

    def g(ci, buf, sem):
        return pltpu.make_async_copy(hp_hbm.at[srcv.at[ci]], rows.at[buf], sem)

    g(0, 0, sem0).start()
    g(1, 1, sem1).start()

    def body(k, carry):
        i = k * 2
        g(i, 0, sem0).wait()
        pltpu.sync_copy(rows.at[0], acc.at[dstv.at[i]], add=True)

        @pl.when(i + 2 < CHE)
        def _s0():
            g(i + 2, 0, sem0).start()

        g(i + 1, 1, sem1).wait()
        pltpu.sync_copy(rows.at[1], acc.at[dstv.at[i + 1]], add=True)

        @pl.when(i + 3 < CHE)
        def _s1():
            g(i + 3, 1, sem1).start()

        return carry

    lax.fori_loop(0, CHE // 2, body, 0)
    plsc.subcore_barrier()
    pltpu.sync_copy(acc.at[pl.ds(s * RPT, RPT)], out_hbm.at[w])


@functools.partial(
    pl.kernel, mesh=_mesh, compiler_params=_sc_params,
    out_type=jax.ShapeDtypeStruct((TB, D), jnp.float32),
    scratch_types=[
        pltpu.VMEM((TB // NW,), jnp.int32),
        pltpu.VMEM((TB // NW, D), jnp.float32),
        pltpu.SemaphoreType.DMA,
    ],
)
def _target_gather(y_hbm, ti_hbm, out_hbm, idxv, rowsv, sem):
    c = lax.axis_index("c")
    s = lax.axis_index("s")
    w = c * 16 + s
    bw = TB // NW
    pltpu.sync_copy(ti_hbm.at[pl.ds(w * bw, bw)], idxv)
    cp = pltpu.make_async_copy(y_hbm.at[idxv], rowsv, sem)
    cp.start()
    cp.wait()
    pltpu.sync_copy(rowsv, out_hbm.at[pl.ds(w * bw, bw)])


def _split(h, tab_ref):
    tab_ref[:NPAD] = h[:, :HD]
    tab_ref[NPAD:] = h[:, HD:]


def _join(tab_ref):
    return jnp.concatenate([tab_ref[:NPAD], tab_ref[NPAD:]], axis=1)


def _prep0_body(degp, xp, w0, htab, dinv):
    d = jnp.sum(degp[0] + degp[1], axis=1)
    r = lax.broadcasted_iota(jnp.int32, (NPAD, 1), 0)
    di = lax.rsqrt(d + 1.0)[:, None]
    di = jnp.where(r < N, di, 0.0)
    di2 = jnp.broadcast_to(di, (NPAD, D))
    _split((xp[...] @ w0[...]) * di2, htab)
    dinv[...] = di2


def _mid_body(p, htab0, dinv, b0, g0, be0, w1, htab1):
    adj = jnp.concatenate([p[0], p[1]], axis=1)
    s0 = (adj + _join(htab0)) * dinv[...] + b0[...]
    sr = s0[:N]
    mean = jnp.mean(sr, axis=0, keepdims=True)
    var = jnp.mean((sr - mean) ** 2, axis=0, keepdims=True)
    h = (s0 - mean) * lax.rsqrt(var + 1e-5) * g0[...] + be0[...]
    h = jnp.maximum(h, 0.0)
    _split((h @ w1[...]) * dinv[...], htab1)


def _fin_body(p, htab1, dinv, b1, wf1, bf1, wf2, bf2, y):
    adj = jnp.concatenate([p[0], p[1]], axis=1)
    out1 = (adj + _join(htab1)) * dinv[...] + b1[...]
    f = jnp.maximum(out1 @ wf1[...] + bf1[...], 0.0)
    y[...] = f @ wf2[...] + bf2[...]


def kernel(x, edge_index, target_index, W0, b0, gamma0, beta0,
           W1, b1, Wf1, bf1, Wf2, bf2):
    f32 = jnp.float32
    src = edge_index[0].astype(jnp.int32)
    dst = edge_index[1].astype(jnp.int32)

    npadd = NW * CHD * CK - E
    padd = N + (jnp.arange(npadd, dtype=jnp.int32) % (NPAD - N))
    dst_wd = jnp.concatenate([dst, padd]).reshape(NW, CHD, CK)

    npade = 16 * CHE * CK - E
    pads = jnp.full((npade,), N, jnp.int32)
    pade = jnp.arange(npade, dtype=jnp.int32) % NPAD
    src_h = jnp.concatenate([src, pads]).reshape(16, CHE, CK)
    src_we = jnp.concatenate([src_h, src_h + NPAD], axis=0)
    dst_h = jnp.concatenate([dst, pade]).reshape(16, CHE, CK)
    dst_we = jnp.concatenate([dst_h, dst_h], axis=0)

    ti = target_index.astype(jnp.int32)
    xpad = jnp.concatenate([x.astype(f32), jnp.zeros((NPAD - N, D), f32)], 0)
    ones16 = jnp.zeros((CK, 16), f32).at[:, 0].set(1.0)
    zdeg = jnp.zeros((RPT, 16), f32)
    zrow = jnp.zeros((RPT, HD), f32)

    degp = _deg_kernel(dst_wd, ones16, zdeg).reshape(2, NPAD, 16)

    htab0, dinv = pl.pallas_call(
        _prep0_body,
        out_shape=[jax.ShapeDtypeStruct((2 * NPAD, HD), f32),
                   jax.ShapeDtypeStruct((NPAD, D), f32)],
    )(degp, xpad, W0)

    p0 = _edge_kernel(htab0, src_we, dst_we, zrow).reshape(2, NPAD, HD)

    htab1 = pl.pallas_call(
        _mid_body,
        out_shape=jax.ShapeDtypeStruct((2 * NPAD, HD), f32),
    )(p0, htab0, dinv, b0.reshape(1, D), gamma0.reshape(1, D),
      beta0.reshape(1, D), W1)

    p1 = _edge_kernel(htab1, src_we, dst_we, zrow).reshape(2, NPAD, HD)

    y = pl.pallas_call(
        _fin_body,
        out_shape=jax.ShapeDtypeStruct((NPAD, D), f32),
    )(p1, htab1, dinv, b1.reshape(1, D), Wf1, bf1.reshape(1, D),
      Wf2, bf2.reshape(1, D))

    return _target_gather(y, ti)


# speedup vs baseline: 1.4179x; 1.0556x over previous
import functools

import jax
import jax.numpy as jnp
from jax import lax
from jax.experimental import pallas as pl
from jax.experimental.pallas import tpu as pltpu
from jax.experimental.pallas import tpu_sc as plsc

N = 10000
D = 128
HD = 64
E = 320000
TB = 4096
NW = 32
CK = 128
CHD = 80
CHE = 158
CK2 = 256
NCHE = 79
NPAD = 10016
RPT = NPAD // 16

_mesh = plsc.VectorSubcoreMesh(core_axis_name="c", subcore_axis_name="s")
_sc_params = pltpu.CompilerParams(use_tc_tiling_on_sc=False)


@functools.partial(
    pl.kernel, mesh=_mesh, compiler_params=_sc_params,
    out_type=jax.ShapeDtypeStruct((NW, RPT, 16), jnp.float32),
    scratch_types=[
        pltpu.VMEM((CHD, CK), jnp.int32),
        pltpu.VMEM((CK, 16), jnp.float32),
        pltpu.VMEM_SHARED((NPAD, 16), jnp.float32),
    ],
)
def _deg_kernel(dst_hbm, ones_hbm, zero_hbm, out_hbm, dstv, onesv, acc):
    c = lax.axis_index("c")
    s = lax.axis_index("s")
    w = c * 16 + s
    pltpu.sync_copy(dst_hbm.at[w], dstv)
    pltpu.sync_copy(ones_hbm, onesv)
    pltpu.sync_copy(zero_hbm, acc.at[pl.ds(s * RPT, RPT)])
    plsc.subcore_barrier()

    def body(i, carry):
        pltpu.sync_copy(onesv, acc.at[dstv.at[i]], add=True)
        return carry

    lax.fori_loop(0, CHD, body, 0)
    plsc.subcore_barrier()
    pltpu.sync_copy(acc.at[pl.ds(s * RPT, RPT)], out_hbm.at[w])


@functools.partial(
    pl.kernel, mesh=_mesh, compiler_params=_sc_params,
    out_type=jax.ShapeDtypeStruct((NW, RPT, HD), jnp.float32),
    scratch_types=[
        pltpu.VMEM((NCHE, CK2), jnp.int32),
        pltpu.VMEM((NCHE, CK2), jnp.int32),
        pltpu.VMEM((2, CK2, HD), jnp.float32),
        pltpu.VMEM_SHARED((NPAD, HD), jnp.float32),
        pltpu.SemaphoreType.DMA,
        pltpu.SemaphoreType.DMA,
    ],
)
def _edge_kernel(hp_hbm, src_hbm, dst_hbm, zero_hbm, out_hbm,
                 srcv, dstv, rows, acc, sem0, sem1):
    c = lax.axis_index("c")
    s = lax.axis_index("s")
    w = c * 16 + s
    pltpu.sync_copy(src_hbm.at[w], srcv)
    pltpu.sync_copy(dst_hbm.at[w], dstv)
    pltpu.sync_copy(zero_hbm, acc.at[pl.ds(s * RPT, RPT)])
    plsc.subcore_barrier()

    def g(ci, buf, sem):
        return pltpu.make_async_copy(hp_hbm.at[srcv.at[ci]], rows.at[buf], sem)

    g(0, 0, sem0).start()
    g(1, 1, sem1).start()

    def body(k, carry):
        i = k * 2
        g(i, 0, sem0).wait()
        pltpu.sync_copy(rows.at[0], acc.at[dstv.at[i]], add=True)

        @pl.when(i + 2 < NCHE)
        def _s0():
            g(i + 2, 0, sem0).start()

        g(i + 1, 1, sem1).wait()
        pltpu.sync_copy(rows.at[1], acc.at[dstv.at[i + 1]], add=True)

        @pl.when(i + 3 < NCHE)
        def _s1():
            g(i + 3, 1, sem1).start()

        return carry

    lax.fori_loop(0, NCHE // 2, body, 0)
    g(NCHE - 1, 0, sem0).wait()
    pltpu.sync_copy(rows.at[0], acc.at[dstv.at[NCHE - 1]], add=True)
    plsc.subcore_barrier()
    pltpu.sync_copy(acc.at[pl.ds(s * RPT, RPT)], out_hbm.at[w])


@functools.partial(
    pl.kernel, mesh=_mesh, compiler_params=_sc_params,
    out_type=jax.ShapeDtypeStruct((TB, D), jnp.float32),
    scratch_types=[
        pltpu.VMEM((TB // NW,), jnp.int32),
        pltpu.VMEM((TB // NW, D), jnp.float32),
        pltpu.SemaphoreType.DMA,
    ],
)
def _target_gather(y_hbm, ti_hbm, out_hbm, idxv, rowsv, sem):
    c = lax.axis_index("c")
    s = lax.axis_index("s")
    w = c * 16 + s
    bw = TB // NW
    pltpu.sync_copy(ti_hbm.at[pl.ds(w * bw, bw)], idxv)
    cp = pltpu.make_async_copy(y_hbm.at[idxv], rowsv, sem)
    cp.start()
    cp.wait()
    pltpu.sync_copy(rowsv, out_hbm.at[pl.ds(w * bw, bw)])


def _split(h, tab_ref):
    tab_ref[:NPAD] = h[:, :HD]
    tab_ref[NPAD:] = h[:, HD:]


def _join(tab_ref):
    return jnp.concatenate([tab_ref[:NPAD], tab_ref[NPAD:]], axis=1)


def _prep0_body(degp, xp, w0, htab, dinv):
    d = jnp.sum(degp[0] + degp[1], axis=1)
    r = lax.broadcasted_iota(jnp.int32, (NPAD, 1), 0)
    di = lax.rsqrt(d + 1.0)[:, None]
    di = jnp.where(r < N, di, 0.0)
    di2 = jnp.broadcast_to(di, (NPAD, D))
    _split((xp[...] @ w0[...]) * di2, htab)
    dinv[...] = di2


def _mid_body(p, htab0, dinv, b0, g0, be0, w1, htab1):
    adj = jnp.concatenate([p[0], p[1]], axis=1)
    s0 = (adj + _join(htab0)) * dinv[...] + b0[...]
    sr = s0[:N]
    mean = jnp.mean(sr, axis=0, keepdims=True)
    var = jnp.mean((sr - mean) ** 2, axis=0, keepdims=True)
    h = (s0 - mean) * lax.rsqrt(var + 1e-5) * g0[...] + be0[...]
    h = jnp.maximum(h, 0.0)
    _split((h @ w1[...]) * dinv[...], htab1)


def _fin_body(p, htab1, dinv, b1, wf1, bf1, wf2, bf2, y):
    adj = jnp.concatenate([p[0], p[1]], axis=1)
    out1 = (adj + _join(htab1)) * dinv[...] + b1[...]
    f = jnp.maximum(out1 @ wf1[...] + bf1[...], 0.0)
    y[...] = f @ wf2[...] + bf2[...]


def kernel(x, edge_index, target_index, W0, b0, gamma0, beta0,
           W1, b1, Wf1, bf1, Wf2, bf2):
    f32 = jnp.float32
    src = edge_index[0].astype(jnp.int32)
    dst = edge_index[1].astype(jnp.int32)

    npadd = NW * CHD * CK - E
    padd = N + (jnp.arange(npadd, dtype=jnp.int32) % (NPAD - N))
    dst_wd = jnp.concatenate([dst, padd]).reshape(NW, CHD, CK)

    npade = 16 * CHE * CK - E
    pads = jnp.full((npade,), N, jnp.int32)
    pade = jnp.arange(npade, dtype=jnp.int32) % NPAD
    src_h = jnp.concatenate([src, pads]).reshape(16, NCHE, CK2)
    src_we = jnp.concatenate([src_h, src_h + NPAD], axis=0)
    dst_h = jnp.concatenate([dst, pade]).reshape(16, NCHE, CK2)
    dst_we = jnp.concatenate([dst_h, dst_h], axis=0)

    ti = target_index.astype(jnp.int32)
    xpad = jnp.concatenate([x.astype(f32), jnp.zeros((NPAD - N, D), f32)], 0)
    ones16 = jnp.zeros((CK, 16), f32).at[:, 0].set(1.0)
    zdeg = jnp.zeros((RPT, 16), f32)
    zrow = jnp.zeros((RPT, HD), f32)

    degp = _deg_kernel(dst_wd, ones16, zdeg).reshape(2, NPAD, 16)

    htab0, dinv = pl.pallas_call(
        _prep0_body,
        out_shape=[jax.ShapeDtypeStruct((2 * NPAD, HD), f32),
                   jax.ShapeDtypeStruct((NPAD, D), f32)],
    )(degp, xpad, W0)

    p0 = _edge_kernel(htab0, src_we, dst_we, zrow).reshape(2, NPAD, HD)

    htab1 = pl.pallas_call(
        _mid_body,
        out_shape=jax.ShapeDtypeStruct((2 * NPAD, HD), f32),
    )(p0, htab0, dinv, b0.reshape(1, D), gamma0.reshape(1, D),
      beta0.reshape(1, D), W1)

    p1 = _edge_kernel(htab1, src_we, dst_we, zrow).reshape(2, NPAD, HD)

    y = pl.pallas_call(
        _fin_body,
        out_shape=jax.ShapeDtypeStruct((NPAD, D), f32),
    )(p1, htab1, dinv, b1.reshape(1, D), Wf1, bf1.reshape(1, D),
      Wf2, bf2.reshape(1, D))

    return _target_gather(y, ti)


# speedup vs baseline: 1.4253x; 1.0052x over previous
import functools

import jax
import jax.numpy as jnp
from jax import lax
from jax.experimental import pallas as pl
from jax.experimental.pallas import tpu as pltpu
from jax.experimental.pallas import tpu_sc as plsc

N = 10000
D = 128
HD = 64
E = 320000
TB = 4096
NW = 32
CK = 128
NCHD = 40
CKD = 256
CHE = 158
CK2 = 256
NCHE = 79
NPAD = 10016
RPT = NPAD // 16

_mesh = plsc.VectorSubcoreMesh(core_axis_name="c", subcore_axis_name="s")
_sc_params = pltpu.CompilerParams(use_tc_tiling_on_sc=False)


@functools.partial(
    pl.kernel, mesh=_mesh, compiler_params=_sc_params,
    out_type=jax.ShapeDtypeStruct((NW, RPT, 16), jnp.float32),
    scratch_types=[
        pltpu.VMEM((NCHD, CKD), jnp.int32),
        pltpu.VMEM((CKD, 16), jnp.float32),
        pltpu.VMEM_SHARED((NPAD, 16), jnp.float32),
    ],
)
def _deg_kernel(dst_hbm, ones_hbm, zero_hbm, out_hbm, dstv, onesv, acc):
    c = lax.axis_index("c")
    s = lax.axis_index("s")
    w = c * 16 + s
    pltpu.sync_copy(dst_hbm.at[w], dstv)
    pltpu.sync_copy(ones_hbm, onesv)
    pltpu.sync_copy(zero_hbm, acc.at[pl.ds(s * RPT, RPT)])
    plsc.subcore_barrier()

    def body(i, carry):
        pltpu.sync_copy(onesv, acc.at[dstv.at[i]], add=True)
        return carry

    lax.fori_loop(0, NCHD, body, 0)
    plsc.subcore_barrier()
    pltpu.sync_copy(acc.at[pl.ds(s * RPT, RPT)], out_hbm.at[w])


@functools.partial(
    pl.kernel, mesh=_mesh, compiler_params=_sc_params,
    out_type=jax.ShapeDtypeStruct((NW, RPT, HD), jnp.float32),
    scratch_types=[
        pltpu.VMEM((NCHE, CK2), jnp.int32),
        pltpu.VMEM((NCHE, CK2), jnp.int32),
        pltpu.VMEM((3, CK2, HD), jnp.float32),
        pltpu.VMEM_SHARED((NPAD, HD), jnp.float32),
        [pltpu.SemaphoreType.DMA] * 3,
        [pltpu.SemaphoreType.DMA] * 3,
    ],
)
def _edge_kernel(hp_hbm, src_hbm, dst_hbm, zero_hbm, out_hbm,
                 srcv, dstv, rows, acc, gsem, ssem):
    c = lax.axis_index("c")
    s = lax.axis_index("s")
    w = c * 16 + s
    pltpu.sync_copy(src_hbm.at[w], srcv)
    pltpu.sync_copy(dst_hbm.at[w], dstv)
    pltpu.sync_copy(zero_hbm, acc.at[pl.ds(s * RPT, RPT)])
    plsc.subcore_barrier()

    def g_desc(ci, b):
        return pltpu.make_async_copy(hp_hbm.at[srcv.at[ci]], rows.at[b],
                                     gsem[b])

    def s_desc(ci, b):
        return pltpu.make_async_copy(rows.at[b], acc.at[dstv.at[ci]],
                                     ssem[b])

    g_desc(0, 0).start()
    g_desc(1, 1).start()

    def step(i, ph):
        g_desc(i, ph).wait()
        pltpu.async_copy(rows.at[ph], acc.at[dstv.at[i]], ssem[ph],
                         add=True)
        k = i + 2
        bk = (ph + 2) % 3

        @pl.when(k < NCHE)
        def _issue():
            @pl.when(k >= 3)
            def _drain():
                s_desc(k - 3, bk).wait()

            g_desc(k, bk).start()

    def body(jj, carry):
        base = jj * 3
        for ph in range(3):
            step(base + ph, ph)
        return carry

    lax.fori_loop(0, NCHE // 3, body, 0)
    for i in range(3 * (NCHE // 3), NCHE):
        step(i, i % 3)
    for i in range(NCHE - 3, NCHE):
        s_desc(i, i % 3).wait()
    plsc.subcore_barrier()
    pltpu.sync_copy(acc.at[pl.ds(s * RPT, RPT)], out_hbm.at[w])


@functools.partial(
    pl.kernel, mesh=_mesh, compiler_params=_sc_params,
    out_type=jax.ShapeDtypeStruct((TB, D), jnp.float32),
    scratch_types=[
        pltpu.VMEM((TB // NW,), jnp.int32),
        pltpu.VMEM((TB // NW, D), jnp.float32),
        pltpu.SemaphoreType.DMA,
    ],
)
def _target_gather(y_hbm, ti_hbm, out_hbm, idxv, rowsv, sem):
    c = lax.axis_index("c")
    s = lax.axis_index("s")
    w = c * 16 + s
    bw = TB // NW
    pltpu.sync_copy(ti_hbm.at[pl.ds(w * bw, bw)], idxv)
    cp = pltpu.make_async_copy(y_hbm.at[idxv], rowsv, sem)
    cp.start()
    cp.wait()
    pltpu.sync_copy(rowsv, out_hbm.at[pl.ds(w * bw, bw)])


def _split(h, tab_ref):
    tab_ref[:NPAD] = h[:, :HD]
    tab_ref[NPAD:] = h[:, HD:]


def _join(tab_ref):
    return jnp.concatenate([tab_ref[:NPAD], tab_ref[NPAD:]], axis=1)


def _mm0_body(xp, w0, out):
    out[...] = xp[...] @ w0[...]


def _scale0_body(degp, mm, htab, dinv):
    d = jnp.sum(degp[0] + degp[1], axis=1)
    r = lax.broadcasted_iota(jnp.int32, (NPAD, 1), 0)
    di = lax.rsqrt(d + 1.0)[:, None]
    di = jnp.where(r < N, di, 0.0)
    di2 = jnp.broadcast_to(di, (NPAD, D))
    _split(mm[...] * di2, htab)
    dinv[...] = di2


def _mid_body(p, htab0, dinv, b0, g0, be0, w1, htab1):
    adj = jnp.concatenate([p[0], p[1]], axis=1)
    s0 = (adj + _join(htab0)) * dinv[...] + b0[...]
    sr = s0[:N]
    mean = jnp.mean(sr, axis=0, keepdims=True)
    var = jnp.mean((sr - mean) ** 2, axis=0, keepdims=True)
    h = (s0 - mean) * lax.rsqrt(var + 1e-5) * g0[...] + be0[...]
    h = jnp.maximum(h, 0.0)
    _split((h @ w1[...]) * dinv[...], htab1)


def _fin_body(p, htab1, dinv, b1, wf1, bf1, wf2, bf2, y):
    adj = jnp.concatenate([p[0], p[1]], axis=1)
    out1 = (adj + _join(htab1)) * dinv[...] + b1[...]
    f = jnp.maximum(out1 @ wf1[...] + bf1[...], 0.0)
    y[...] = f @ wf2[...] + bf2[...]


def kernel(x, edge_index, target_index, W0, b0, gamma0, beta0,
           W1, b1, Wf1, bf1, Wf2, bf2):
    f32 = jnp.float32
    src = edge_index[0].astype(jnp.int32)
    dst = edge_index[1].astype(jnp.int32)

    npadd = NW * NCHD * CKD - E
    padd = N + (jnp.arange(npadd, dtype=jnp.int32) % (NPAD - N))
    dst_wd = jnp.concatenate([dst, padd]).reshape(NW, NCHD, CKD)

    npade = 16 * CHE * CK - E
    pads = jnp.full((npade,), N, jnp.int32)
    pade = jnp.arange(npade, dtype=jnp.int32) % NPAD
    src_h = jnp.concatenate([src, pads]).reshape(16, NCHE, CK2)
    src_we = jnp.concatenate([src_h, src_h + NPAD], axis=0)
    dst_h = jnp.concatenate([dst, pade]).reshape(16, NCHE, CK2)
    dst_we = jnp.concatenate([dst_h, dst_h], axis=0)

    ti = target_index.astype(jnp.int32)
    xpad = jnp.concatenate([x.astype(f32), jnp.zeros((NPAD - N, D), f32)], 0)
    ones16 = jnp.zeros((CKD, 16), f32).at[:, 0].set(1.0)
    zdeg = jnp.zeros((RPT, 16), f32)
    zrow = jnp.zeros((RPT, HD), f32)

    mm0 = pl.pallas_call(
        _mm0_body,
        out_shape=jax.ShapeDtypeStruct((NPAD, D), f32),
    )(xpad, W0)

    degp = _deg_kernel(dst_wd, ones16, zdeg).reshape(2, NPAD, 16)

    htab0, dinv = pl.pallas_call(
        _scale0_body,
        out_shape=[jax.ShapeDtypeStruct((2 * NPAD, HD), f32),
                   jax.ShapeDtypeStruct((NPAD, D), f32)],
    )(degp, mm0)

    p0 = _edge_kernel(htab0, src_we, dst_we, zrow).reshape(2, NPAD, HD)

    htab1 = pl.pallas_call(
        _mid_body,
        out_shape=jax.ShapeDtypeStruct((2 * NPAD, HD), f32),
    )(p0, htab0, dinv, b0.reshape(1, D), gamma0.reshape(1, D),
      beta0.reshape(1, D), W1)

    p1 = _edge_kernel(htab1, src_we, dst_we, zrow).reshape(2, NPAD, HD)

    y = pl.pallas_call(
        _fin_body,
        out_shape=jax.ShapeDtypeStruct((NPAD, D), f32),
    )(p1, htab1, dinv, b1.reshape(1, D), Wf1, bf1.reshape(1, D),
      Wf2, bf2.reshape(1, D))

    return _target_gather(y, ti)


# speedup vs baseline: 1.4766x; 1.0361x over previous
import functools

import jax
import jax.numpy as jnp
from jax import lax
from jax.experimental import pallas as pl
from jax.experimental.pallas import tpu as pltpu
from jax.experimental.pallas import tpu_sc as plsc

N = 10000
D = 128
HD = 64
E = 320000
TB = 4096
NW = 32
CK = 128
NCHD = 20
CKD = 512
CHE = 158
CK2 = 256
NCHE = 79
NPAD = 10016
RPT = NPAD // 16

_mesh = plsc.VectorSubcoreMesh(core_axis_name="c", subcore_axis_name="s")
_sc_params = pltpu.CompilerParams(use_tc_tiling_on_sc=False)


@functools.partial(
    pl.kernel, mesh=_mesh, compiler_params=_sc_params,
    out_type=jax.ShapeDtypeStruct((NW, RPT, 16), jnp.float32),
    scratch_types=[
        pltpu.VMEM((NCHD, CKD), jnp.int32),
        pltpu.VMEM((CKD, 16), jnp.float32),
        pltpu.VMEM_SHARED((NPAD, 16), jnp.float32),
    ],
)
def _deg_kernel(dst_hbm, ones_hbm, zero_hbm, out_hbm, dstv, onesv, acc):
    c = lax.axis_index("c")
    s = lax.axis_index("s")
    w = c * 16 + s
    pltpu.sync_copy(dst_hbm.at[w], dstv)
    pltpu.sync_copy(ones_hbm, onesv)
    pltpu.sync_copy(zero_hbm, acc.at[pl.ds(s * RPT, RPT)])
    plsc.subcore_barrier()

    def body(i, carry):
        pltpu.sync_copy(onesv, acc.at[dstv.at[i]], add=True)
        return carry

    lax.fori_loop(0, NCHD, body, 0)
    plsc.subcore_barrier()
    pltpu.sync_copy(acc.at[pl.ds(s * RPT, RPT)], out_hbm.at[w])


@functools.partial(
    pl.kernel, mesh=_mesh, compiler_params=_sc_params,
    out_type=jax.ShapeDtypeStruct((NW, RPT, HD), jnp.float32),
    scratch_types=[
        pltpu.VMEM((NCHE, CK2), jnp.int32),
        pltpu.VMEM((NCHE, CK2), jnp.int32),
        pltpu.VMEM((3, CK2, HD), jnp.float32),
        pltpu.VMEM_SHARED((NPAD, HD), jnp.float32),
        [pltpu.SemaphoreType.DMA] * 3,
        [pltpu.SemaphoreType.DMA] * 3,
    ],
)
def _edge_kernel(hp_hbm, src_hbm, dst_hbm, zero_hbm, out_hbm,
                 srcv, dstv, rows, acc, gsem, ssem):
    c = lax.axis_index("c")
    s = lax.axis_index("s")
    w = c * 16 + s
    pltpu.sync_copy(src_hbm.at[w], srcv)
    pltpu.sync_copy(dst_hbm.at[w], dstv)
    pltpu.sync_copy(zero_hbm, acc.at[pl.ds(s * RPT, RPT)])
    plsc.subcore_barrier()

    def g_desc(ci, b):
        return pltpu.make_async_copy(hp_hbm.at[srcv.at[ci]], rows.at[b],
                                     gsem[b])

    def s_desc(ci, b):
        return pltpu.make_async_copy(rows.at[b], acc.at[dstv.at[ci]],
                                     ssem[b])

    g_desc(0, 0).start()
    g_desc(1, 1).start()

    def step(i, ph):
        g_desc(i, ph).wait()
        pltpu.async_copy(rows.at[ph], acc.at[dstv.at[i]], ssem[ph],
                         add=True)
        k = i + 2
        bk = (ph + 2) % 3

        @pl.when(k < NCHE)
        def _issue():
            @pl.when(k >= 3)
            def _drain():
                s_desc(k - 3, bk).wait()

            g_desc(k, bk).start()

    def body(jj, carry):
        base = jj * 3
        for ph in range(3):
            step(base + ph, ph)
        return carry

    lax.fori_loop(0, NCHE // 3, body, 0)
    for i in range(3 * (NCHE // 3), NCHE):
        step(i, i % 3)
    for i in range(NCHE - 3, NCHE):
        s_desc(i, i % 3).wait()
    plsc.subcore_barrier()
    pltpu.sync_copy(acc.at[pl.ds(s * RPT, RPT)], out_hbm.at[w])


@functools.partial(
    pl.kernel, mesh=_mesh, compiler_params=_sc_params,
    out_type=jax.ShapeDtypeStruct((TB, D), jnp.float32),
    scratch_types=[
        pltpu.VMEM((TB // NW,), jnp.int32),
        pltpu.VMEM((TB // NW, D), jnp.float32),
        pltpu.SemaphoreType.DMA,
    ],
)
def _target_gather(y_hbm, ti_hbm, out_hbm, idxv, rowsv, sem):
    c = lax.axis_index("c")
    s = lax.axis_index("s")
    w = c * 16 + s
    bw = TB // NW
    pltpu.sync_copy(ti_hbm.at[pl.ds(w * bw, bw)], idxv)
    cp = pltpu.make_async_copy(y_hbm.at[idxv], rowsv, sem)
    cp.start()
    cp.wait()
    pltpu.sync_copy(rowsv, out_hbm.at[pl.ds(w * bw, bw)])


def _split(h, tab_ref):
    tab_ref[:NPAD] = h[:, :HD]
    tab_ref[NPAD:] = h[:, HD:]


def _join(tab_ref):
    return jnp.concatenate([tab_ref[:NPAD], tab_ref[NPAD:]], axis=1)


def _mm0_body(xp, w0, out):
    out[...] = xp[...] @ w0[...]


def _scale0_body(degp, mm, htab, dinv):
    d = jnp.sum(degp[0] + degp[1], axis=1)
    r = lax.broadcasted_iota(jnp.int32, (NPAD, 1), 0)
    di = lax.rsqrt(d + 1.0)[:, None]
    di = jnp.where(r < N, di, 0.0)
    di2 = jnp.broadcast_to(di, (NPAD, D))
    _split(mm[...] * di2, htab)
    dinv[...] = di2


def _mid_body(p, htab0, dinv, b0, g0, be0, w1, htab1):
    adj = jnp.concatenate([p[0], p[1]], axis=1)
    s0 = (adj + _join(htab0)) * dinv[...] + b0[...]
    sr = s0[:N]
    mean = jnp.mean(sr, axis=0, keepdims=True)
    var = jnp.mean((sr - mean) ** 2, axis=0, keepdims=True)
    h = (s0 - mean) * lax.rsqrt(var + 1e-5) * g0[...] + be0[...]
    h = jnp.maximum(h, 0.0)
    _split((h @ w1[...]) * dinv[...], htab1)


def _fin_body(p, htab1, dinv, b1, wf1, bf1, wf2, bf2, y):
    adj = jnp.concatenate([p[0], p[1]], axis=1)
    out1 = (adj + _join(htab1)) * dinv[...] + b1[...]
    f = jnp.maximum(out1 @ wf1[...] + bf1[...], 0.0)
    y[...] = f @ wf2[...] + bf2[...]


def kernel(x, edge_index, target_index, W0, b0, gamma0, beta0,
           W1, b1, Wf1, bf1, Wf2, bf2):
    f32 = jnp.float32
    src = edge_index[0].astype(jnp.int32)
    dst = edge_index[1].astype(jnp.int32)

    npadd = NW * NCHD * CKD - E
    padd = N + (jnp.arange(npadd, dtype=jnp.int32) % (NPAD - N))
    dst_wd = jnp.concatenate([dst, padd]).reshape(NW, NCHD, CKD)

    npade = 16 * CHE * CK - E
    pads = jnp.full((npade,), N, jnp.int32)
    pade = jnp.arange(npade, dtype=jnp.int32) % NPAD
    src_h = jnp.concatenate([src, pads]).reshape(16, NCHE, CK2)
    src_we = jnp.concatenate([src_h, src_h + NPAD], axis=0)
    dst_h = jnp.concatenate([dst, pade]).reshape(16, NCHE, CK2)
    dst_we = jnp.concatenate([dst_h, dst_h], axis=0)

    ti = target_index.astype(jnp.int32)
    xpad = jnp.concatenate([x.astype(f32), jnp.zeros((NPAD - N, D), f32)], 0)
    ones16 = jnp.zeros((CKD, 16), f32).at[:, 0].set(1.0)
    zdeg = jnp.zeros((RPT, 16), f32)
    zrow = jnp.zeros((RPT, HD), f32)

    mm0 = pl.pallas_call(
        _mm0_body,
        out_shape=jax.ShapeDtypeStruct((NPAD, D), f32),
    )(xpad, W0)

    degp = _deg_kernel(dst_wd, ones16, zdeg).reshape(2, NPAD, 16)

    htab0, dinv = pl.pallas_call(
        _scale0_body,
        out_shape=[jax.ShapeDtypeStruct((2 * NPAD, HD), f32),
                   jax.ShapeDtypeStruct((NPAD, D), f32)],
    )(degp, mm0)

    p0 = _edge_kernel(htab0, src_we, dst_we, zrow).reshape(2, NPAD, HD)

    htab1 = pl.pallas_call(
        _mid_body,
        out_shape=jax.ShapeDtypeStruct((2 * NPAD, HD), f32),
    )(p0, htab0, dinv, b0.reshape(1, D), gamma0.reshape(1, D),
      beta0.reshape(1, D), W1)

    p1 = _edge_kernel(htab1, src_we, dst_we, zrow).reshape(2, NPAD, HD)

    y = pl.pallas_call(
        _fin_body,
        out_shape=jax.ShapeDtypeStruct((NPAD, D), f32),
    )(p1, htab1, dinv, b1.reshape(1, D), Wf1, bf1.reshape(1, D),
      Wf2, bf2.reshape(1, D))

    return _target_gather(y, ti)
